# baseline (speedup 1.0000x reference)
import functools

import jax
import jax.numpy as jnp
from jax import lax
from jax.experimental import pallas as pl
from jax.experimental.pallas import tpu as pltpu
from jax.experimental.pallas import tpu_sc as plsc

_N_FRAMES = 8


def kernel(x, number):
    B, F, D = x.shape
    n = _N_FRAMES
    step = (F // (number + 1)).astype(jnp.int32)
    ids = jnp.arange(1, n + 1, dtype=jnp.int32) * step
    flat_ids = (
        jnp.arange(B, dtype=jnp.int32)[:, None] * F + ids[None, :]
    ).reshape(B * n)
    x2d = x.reshape(B * F, D)

    info = plsc.get_sparse_core_info()
    NC, NS = info.num_cores, info.num_subcores
    NW = NC * NS
    rows_total = B * n
    assert rows_total % NW == 0
    rpw = rows_total // NW
    assert (rpw * 1) % 8 == 0 or rpw % 8 == 0

    mesh = plsc.VectorSubcoreMesh(core_axis_name="c", subcore_axis_name="s")

    @functools.partial(
        pl.kernel,
        mesh=mesh,
        out_type=jax.ShapeDtypeStruct((rows_total, D), x.dtype),
        scratch_types=[
            pltpu.VMEM((rpw,), jnp.int32),
            pltpu.VMEM((rpw, D), x.dtype),
            pltpu.SemaphoreType.DMA,
        ],
    )
    def gather_rows(table_hbm, idx_hbm, out_hbm, idx_v, rows_v, sem):
        wid = lax.axis_index("s") * NC + lax.axis_index("c")
        base = wid * rpw
        pltpu.sync_copy(idx_hbm.at[pl.ds(base, rpw)], idx_v)
        pltpu.async_copy(table_hbm.at[idx_v], rows_v, sem).wait()
        pltpu.sync_copy(rows_v, out_hbm.at[pl.ds(base, rpw)])

    return gather_rows(x2d, flat_ids).reshape(B, n, D)
